# baseline (speedup 1.0000x reference)
import functools
import jax
import jax.numpy as jnp
from jax import lax
from jax.experimental import pallas as pl
from jax.experimental.pallas import tpu as pltpu, tpu_sc as plsc

BS = 512


def _add_body(idxx_ref, ex_ref, posyz_ref, x_ref, out_ref):
    d3 = ex_ref.shape[-1]
    nrow = ex_ref.shape[0]
    bs = idxx_ref.shape[-1]
    ix = idxx_ref[0, 0]
    iot = lax.broadcasted_iota(jnp.int32, (bs, nrow), 1)
    oh = (iot == ix[:, None]).astype(jnp.float32)
    posx = jnp.dot(oh, ex_ref[...], preferred_element_type=jnp.float32)
    out_ref[:, :, :d3] = x_ref[:, :, :d3] + posx[None]
    out_ref[:, :, d3:] = (
        x_ref[:, :, d3:] + posyz_ref[...].astype(jnp.float32)[None])


def _make_sc_gather(S, d3):
    info = plsc.get_sparse_core_info()
    nw = info.num_cores * info.num_subcores
    spw = S // nw
    mesh = plsc.VectorSubcoreMesh(core_axis_name="c", subcore_axis_name="s")

    @functools.partial(
        pl.kernel, mesh=mesh,
        out_type=jax.ShapeDtypeStruct((S, d3), jnp.int32),
        scratch_types=[
            pltpu.VMEM((spw,), jnp.int32),
            pltpu.VMEM((spw, d3), jnp.int32),
            pltpu.SemaphoreType.DMA,
        ],
    )
    def sc_gather(yztab_hbm, idx_hbm, posyz_hbm, idx_v, ryz_v, sem):
        wid = lax.axis_index("s") * info.num_cores + lax.axis_index("c")
        base = wid * spw
        pltpu.sync_copy(idx_hbm.at[pl.ds(base, spw)], idx_v)
        pltpu.async_copy(yztab_hbm.at[idx_v], ryz_v, sem).wait()
        pltpu.sync_copy(ryz_v, posyz_hbm.at[pl.ds(base, spw)])

    return sc_gather


def kernel(x, src_tgt, emb_x, emb_y, emb_z, src_pos_x, src_pos_y, src_pos_z):
    B, S, D = x.shape
    d3 = emb_x.shape[1]
    nx, ny, nz = emb_x.shape[0], emb_y.shape[0], emb_z.shape[0]

    is_src = (src_tgt != 0)
    sx = jnp.concatenate([jnp.array([nx - 1], jnp.int32), src_pos_x])[:S]
    sy = jnp.concatenate([jnp.array([ny - 1], jnp.int32), src_pos_y])[:S]
    sz = jnp.concatenate([jnp.array([nz - 1], jnp.int32), src_pos_z])[:S]
    px = jnp.where(is_src, src_pos_x, sx)
    py = jnp.where(is_src, src_pos_y, sy)
    pz = jnp.where(is_src, src_pos_z, sz)

    yztab = jnp.concatenate(
        [jnp.repeat(emb_y, nz, axis=0), jnp.tile(emb_z, (ny, 1))],
        axis=1).astype(jnp.bfloat16)
    yztab_i32 = jax.lax.bitcast_convert_type(
        yztab.reshape(ny * nz, d3, 2), jnp.int32)
    posyz_i32 = _make_sc_gather(S, d3)(yztab_i32, py * nz + pz)
    posyz = jax.lax.bitcast_convert_type(
        posyz_i32, jnp.bfloat16).reshape(S, 2 * d3)

    nxp = (nx + 7) // 8 * 8
    ex = jnp.zeros((nxp, d3), jnp.float32).at[:nx].set(emb_x)

    nb = S // BS
    idxx = px.reshape(nb, 1, BS)
    out = pl.pallas_call(
        _add_body,
        grid=(nb,),
        in_specs=[
            pl.BlockSpec((1, 1, BS), lambda i: (i, 0, 0)),
            pl.BlockSpec((nxp, d3), lambda i: (0, 0)),
            pl.BlockSpec((BS, 2 * d3), lambda i: (i, 0)),
            pl.BlockSpec((B, BS, D), lambda i: (0, i, 0)),
        ],
        out_specs=pl.BlockSpec((B, BS, D), lambda i: (0, i, 0)),
        out_shape=jax.ShapeDtypeStruct((B, S, D), jnp.float32),
    )(idxx, ex, posyz, x)
    return out


# speedup vs baseline: 1.5498x; 1.5498x over previous
import functools
import jax
import jax.numpy as jnp
from jax import lax
from jax.experimental import pallas as pl
from jax.experimental.pallas import tpu as pltpu, tpu_sc as plsc

BS = 512


def _addx_body(idxx_ref, ex_ref, x_ref, out_ref):
    nrow = ex_ref.shape[0]
    bs = idxx_ref.shape[-1]
    ix = idxx_ref[0, 0]
    iot = lax.broadcasted_iota(jnp.int32, (bs, nrow), 1)
    oh = (iot == ix[:, None]).astype(jnp.float32)
    posx = jnp.dot(oh, ex_ref[...], preferred_element_type=jnp.float32)
    out_ref[...] = x_ref[...] + posx[None]


def _addyz_body(alias_ref, posyz_ref, x_ref, out_ref):
    del alias_ref
    out_ref[...] = x_ref[...] + posyz_ref[...][None]


def _make_sc_gather(S, d3):
    info = plsc.get_sparse_core_info()
    nw = info.num_cores * info.num_subcores
    spw = S // nw
    mesh = plsc.VectorSubcoreMesh(core_axis_name="c", subcore_axis_name="s")

    @functools.partial(
        pl.kernel, mesh=mesh,
        out_type=jax.ShapeDtypeStruct((S, 2 * d3), jnp.float32),
        scratch_types=[
            pltpu.VMEM((spw,), jnp.int32),
            pltpu.VMEM((spw, 2 * d3), jnp.float32),
            pltpu.SemaphoreType.DMA,
        ],
    )
    def sc_gather(yztab_hbm, idx_hbm, posyz_hbm, idx_v, ryz_v, sem):
        wid = lax.axis_index("s") * info.num_cores + lax.axis_index("c")
        base = wid * spw
        pltpu.sync_copy(idx_hbm.at[pl.ds(base, spw)], idx_v)
        pltpu.async_copy(yztab_hbm.at[idx_v], ryz_v, sem).wait()
        pltpu.sync_copy(ryz_v, posyz_hbm.at[pl.ds(base, spw)])

    return sc_gather


def kernel(x, src_tgt, emb_x, emb_y, emb_z, src_pos_x, src_pos_y, src_pos_z):
    B, S, D = x.shape
    d3 = emb_x.shape[1]
    nx, ny, nz = emb_x.shape[0], emb_y.shape[0], emb_z.shape[0]

    is_src = (src_tgt != 0)
    sx = jnp.concatenate([jnp.array([nx - 1], jnp.int32), src_pos_x])[:S]
    sy = jnp.concatenate([jnp.array([ny - 1], jnp.int32), src_pos_y])[:S]
    sz = jnp.concatenate([jnp.array([nz - 1], jnp.int32), src_pos_z])[:S]
    px = jnp.where(is_src, src_pos_x, sx)
    py = jnp.where(is_src, src_pos_y, sy)
    pz = jnp.where(is_src, src_pos_z, sz)

    yztab = jnp.concatenate(
        [jnp.repeat(emb_y, nz, axis=0), jnp.tile(emb_z, (ny, 1))], axis=1)
    posyz = _make_sc_gather(S, d3)(yztab, py * nz + pz)

    nxp = (nx + 7) // 8 * 8
    ex = jnp.zeros((nxp, d3), jnp.float32).at[:nx].set(emb_x)

    nb = S // BS
    idxx = px.reshape(nb, 1, BS)

    out1 = pl.pallas_call(
        _addx_body,
        grid=(nb,),
        in_specs=[
            pl.BlockSpec((1, 1, BS), lambda i: (i, 0, 0)),
            pl.BlockSpec((nxp, d3), lambda i: (0, 0)),
            pl.BlockSpec((B, BS, d3), lambda i: (0, i, 0)),
        ],
        out_specs=pl.BlockSpec((B, BS, d3), lambda i: (0, i, 0)),
        out_shape=jax.ShapeDtypeStruct((B, S, D), jnp.float32),
    )(idxx, ex, x)

    out = pl.pallas_call(
        _addyz_body,
        grid=(nb, 2),
        in_specs=[
            pl.BlockSpec(memory_space=pl.ANY),
            pl.BlockSpec((BS, d3), lambda i, j: (i, j)),
            pl.BlockSpec((B, BS, d3), lambda i, j: (0, i, j + 1)),
        ],
        out_specs=pl.BlockSpec((B, BS, d3), lambda i, j: (0, i, j + 1)),
        out_shape=jax.ShapeDtypeStruct((B, S, D), jnp.float32),
        input_output_aliases={0: 0},
    )(out1, posyz, x)
    return out


# speedup vs baseline: 1.6363x; 1.0558x over previous
import functools
import jax
import jax.numpy as jnp
from jax import lax
from jax.experimental import pallas as pl
from jax.experimental.pallas import tpu as pltpu, tpu_sc as plsc

BS = 512


def _add_body(idxx_ref, ex_ref, posyz_ref, x_ref, out_ref):
    d3 = ex_ref.shape[-1]
    nrow = ex_ref.shape[0]
    bs = idxx_ref.shape[-1]
    ix = idxx_ref[0, 0]
    iot = lax.broadcasted_iota(jnp.int32, (bs, nrow), 1)
    oh = (iot == ix[:, None]).astype(jnp.float32)
    posx = jnp.dot(oh, ex_ref[...], preferred_element_type=jnp.float32)
    out_ref[:, :, :d3] = x_ref[:, :, :d3] + posx[None]
    out_ref[:, :, d3:] = x_ref[:, :, d3:] + posyz_ref[...][None]


def _make_sc_gather(S, d3):
    info = plsc.get_sparse_core_info()
    nw = info.num_cores * info.num_subcores
    spw = S // nw
    mesh = plsc.VectorSubcoreMesh(core_axis_name="c", subcore_axis_name="s")

    @functools.partial(
        pl.kernel, mesh=mesh,
        out_type=jax.ShapeDtypeStruct((S, 2 * d3), jnp.float32),
        scratch_types=[
            pltpu.VMEM((spw,), jnp.int32),
            pltpu.VMEM((spw, 2 * d3), jnp.float32),
            pltpu.SemaphoreType.DMA,
        ],
    )
    def sc_gather(yztab_hbm, idx_hbm, posyz_hbm, idx_v, ryz_v, sem):
        wid = lax.axis_index("s") * info.num_cores + lax.axis_index("c")
        base = wid * spw
        pltpu.sync_copy(idx_hbm.at[pl.ds(base, spw)], idx_v)
        pltpu.async_copy(yztab_hbm.at[idx_v], ryz_v, sem).wait()
        pltpu.sync_copy(ryz_v, posyz_hbm.at[pl.ds(base, spw)])

    return sc_gather


def kernel(x, src_tgt, emb_x, emb_y, emb_z, src_pos_x, src_pos_y, src_pos_z):
    B, S, D = x.shape
    d3 = emb_x.shape[1]
    nx, ny, nz = emb_x.shape[0], emb_y.shape[0], emb_z.shape[0]

    is_src = (src_tgt != 0)
    sx = jnp.concatenate([jnp.array([nx - 1], jnp.int32), src_pos_x])[:S]
    sy = jnp.concatenate([jnp.array([ny - 1], jnp.int32), src_pos_y])[:S]
    sz = jnp.concatenate([jnp.array([nz - 1], jnp.int32), src_pos_z])[:S]
    px = jnp.where(is_src, src_pos_x, sx)
    py = jnp.where(is_src, src_pos_y, sy)
    pz = jnp.where(is_src, src_pos_z, sz)

    yztab = jnp.concatenate(
        [jnp.repeat(emb_y, nz, axis=0), jnp.tile(emb_z, (ny, 1))], axis=1)
    posyz = _make_sc_gather(S, d3)(yztab, py * nz + pz)

    nxp = (nx + 7) // 8 * 8
    ex = jnp.zeros((nxp, d3), jnp.float32).at[:nx].set(emb_x)

    nb = S // BS
    idxx = px.reshape(nb, 1, BS)
    out = pl.pallas_call(
        _add_body,
        grid=(nb,),
        in_specs=[
            pl.BlockSpec((1, 1, BS), lambda i: (i, 0, 0)),
            pl.BlockSpec((nxp, d3), lambda i: (0, 0)),
            pl.BlockSpec((BS, 2 * d3), lambda i: (i, 0)),
            pl.BlockSpec((B, BS, D), lambda i: (0, i, 0)),
        ],
        out_specs=pl.BlockSpec((B, BS, D), lambda i: (0, i, 0)),
        out_shape=jax.ShapeDtypeStruct((B, S, D), jnp.float32),
    )(idxx, ex, posyz, x)
    return out
